# baseline (device time: 254683 ns/iter reference)
import jax
import jax.numpy as jnp
from jax import lax
from jax.experimental import pallas as pl
from jax.experimental.pallas import tpu as pltpu

N_DEV = 32
E_PER = 2
N_EXP = N_DEV * E_PER


def kernel(x, router_W, route_idx, expert_W):
    n_tok, d_model = x.shape
    _, _, d_ff = expert_W.shape

    def body(x_ref, rw_ref, idx_ref, ew_ref, out_ref,
             comm_ref, send_sems, recv_sems):
        my = lax.axis_index("i")
        left = lax.rem(my - 1 + N_DEV, N_DEV)
        right = lax.rem(my + 1, N_DEV)

        barrier = pltpu.get_barrier_semaphore()
        for nbr in (left, right):
            pl.semaphore_signal(barrier, inc=1, device_id=(nbr,),
                                device_id_type=pl.DeviceIdType.MESH)
        pl.semaphore_wait(barrier, 2)

        xf = x_ref[:, :]
        scores = lax.dot_general(
            xf, rw_ref[:, :], (((1,), (0,)), ((), ())),
            preferred_element_type=jnp.float32)
        p = jnp.exp(scores - jnp.max(scores, axis=1, keepdims=True))
        eids = lax.broadcasted_iota(jnp.int32, (n_tok, N_EXP), 1)
        m0 = (eids == idx_ref[:, 0:1]).astype(jnp.float32)
        m1 = (eids == idx_ref[:, 1:2]).astype(jnp.float32)
        g0 = jnp.sum(p * m0, axis=1, keepdims=True)
        g1 = jnp.sum(p * m1, axis=1, keepdims=True)
        inv = 1.0 / (g0 + g1)
        gates = m0 * (g0 * inv) + m1 * (g1 * inv)

        xb = xf.astype(jnp.bfloat16)

        def contrib(w_pair, origin, acc):
            for k in range(E_PER):
                e = origin * E_PER + k
                col = jnp.sum(
                    gates * (eids == e).astype(jnp.float32),
                    axis=1, keepdims=True)
                y = lax.dot_general(
                    xb, w_pair[k], (((1,), (0,)), ((), ())),
                    preferred_element_type=jnp.float32)
                acc = acc + col * y
            return acc

        wb = ew_ref[:, :, :].astype(jnp.bfloat16)
        comm_ref[0, :, :, :] = wb
        acc = contrib(wb, my, acc=jnp.zeros((n_tok, d_ff), jnp.float32))

        for h in range(N_DEV - 1):
            s_slot = h % 2
            r_slot = (h + 1) % 2
            rdma = pltpu.make_async_remote_copy(
                src_ref=comm_ref.at[s_slot],
                dst_ref=comm_ref.at[r_slot],
                send_sem=send_sems.at[h],
                recv_sem=recv_sems.at[h],
                device_id=(right,),
                device_id_type=pl.DeviceIdType.MESH,
            )
            rdma.start()
            rdma.wait()
            origin = lax.rem(my - (h + 1) + N_DEV, N_DEV)
            acc = contrib(comm_ref[r_slot], origin, acc)

        out_ref[:, :] = acc

    return pl.pallas_call(
        body,
        out_shape=jax.ShapeDtypeStruct((n_tok, d_ff), jnp.float32),
        in_specs=[pl.BlockSpec(memory_space=pltpu.VMEM)] * 4,
        out_specs=pl.BlockSpec(memory_space=pltpu.VMEM),
        scratch_shapes=[
            pltpu.VMEM((2, E_PER, d_model, d_ff), jnp.bfloat16),
            pltpu.SemaphoreType.DMA((N_DEV - 1,)),
            pltpu.SemaphoreType.DMA((N_DEV - 1,)),
        ],
        compiler_params=pltpu.CompilerParams(collective_id=0),
    )(x, router_W, route_idx, expert_W)


# device time: 103656 ns/iter; 2.4570x vs baseline; 2.4570x over previous
import jax
import jax.numpy as jnp
from jax import lax
from jax.experimental import pallas as pl
from jax.experimental.pallas import tpu as pltpu

N_DEV = 32
E_PER = 2
N_EXP = N_DEV * E_PER
N_SUPER = 16
CW_H = 8
CCW_H = 7
DUP = 8


def kernel(x, router_W, route_idx, expert_W):
    n_tok, d_model = x.shape
    _, _, d_ff = expert_W.shape

    def body(x_ref, rw_ref, idx_ref, ew_ref, out_ref,
             pair_ref, xpair_ref, cw_ref, ccw_ref, xcw_ref, xccw_ref,
             xpair_s, xpair_r, cw_s, cw_r, ccw_s, ccw_r,
             xcw_s, xcw_r, xccw_s, xccw_r):
        my = lax.axis_index("i")
        z0 = my // 8
        r0 = my % 8
        y0 = r0 // 2
        even0 = (y0 % 2) == 0
        x0 = jnp.where(even0, r0 - 2 * y0, 2 * y0 + 1 - r0)

        def logof(x_, y_, z_):
            ev = (y_ % 2) == 0
            return 8 * z_ + jnp.where(ev, 2 * y_ + x_, 2 * y_ + 1 - x_)

        def pos_of(y_, z_):
            return jnp.where(
                y_ == 0, z_,
                jnp.where(
                    y_ == 1, jnp.where(z_ == 0, 15, 7 - z_),
                    jnp.where(
                        y_ == 2, jnp.where(z_ == 0, 14, 6 + z_),
                        13 - z_)))

        def sc_yz(p_):
            q = lax.rem(p_ + 2 * N_SUPER, N_SUPER)
            y_ = jnp.where(
                q <= 3, 0,
                jnp.where(q <= 6, 1,
                          jnp.where(q <= 9, 2,
                                    jnp.where(q <= 13, 3,
                                              jnp.where(q == 14, 2, 1)))))
            z_ = jnp.where(
                q <= 3, q,
                jnp.where(q <= 6, 7 - q,
                          jnp.where(q <= 9, q - 6,
                                    jnp.where(q <= 13, 13 - q, 0))))
            return y_, z_

        P = pos_of(y0, z0)
        pm = logof(1 - x0, y0, z0)

        def lane_dev(p_):
            y_, z_ = sc_yz(p_)
            return logof(x0, y_, z_)

        def pair_dev(p_):
            y_, z_ = sc_yz(p_)
            return logof(1 - x0, y_, z_)

        nxt = lane_dev(P + 1)
        prv = lane_dev(P - 1)

        barrier = pltpu.get_barrier_semaphore()
        for nbr in (nxt, prv, pm):
            pl.semaphore_signal(barrier, inc=1, device_id=(nbr,),
                                device_id_type=pl.DeviceIdType.MESH)
        pl.semaphore_wait(barrier, 3)

        started = []

        pair_ref[:, :, :] = ew_ref[:, :, :].astype(jnp.bfloat16)

        xpair_rdma = []
        for c in range(E_PER):
            d = pltpu.make_async_remote_copy(
                src_ref=pair_ref.at[c], dst_ref=xpair_ref.at[c],
                send_sem=xpair_s.at[c], recv_sem=xpair_r.at[c],
                device_id=(pm,), device_id_type=pl.DeviceIdType.MESH)
            d.start()
            xpair_rdma.append(d)
            started.append(d)

        def lane_hop(dref, ss, rr, h, c, target, src):
            return pltpu.make_async_remote_copy(
                src_ref=src, dst_ref=dref.at[h, c],
                send_sem=ss.at[h, c], recv_sem=rr.at[h, c],
                device_id=(target,), device_id_type=pl.DeviceIdType.MESH)

        def xrelay(lref, xref, ss, rr, h, c):
            return pltpu.make_async_remote_copy(
                src_ref=lref.at[h, c], dst_ref=xref.at[h, c],
                send_sem=ss.at[h, c], recv_sem=rr.at[h, c],
                device_id=(pm,), device_id_type=pl.DeviceIdType.MESH)

        cw_rdma = {}
        ccw_rdma = {}
        for c in range(E_PER):
            cw_rdma[(0, c)] = lane_hop(cw_ref, cw_s, cw_r, 0, c, nxt,
                                       pair_ref.at[c])
            cw_rdma[(0, c)].start()
            ccw_rdma[(0, c)] = lane_hop(ccw_ref, ccw_s, ccw_r, 0, c, prv,
                                        pair_ref.at[c])
            ccw_rdma[(0, c)].start()
            started += [cw_rdma[(0, c)], ccw_rdma[(0, c)]]

        xf = x_ref[:, :]
        scores = lax.dot_general(
            xf, rw_ref[:, :], (((1,), (0,)), ((), ())),
            preferred_element_type=jnp.float32)
        p = jnp.exp(scores - jnp.max(scores, axis=1, keepdims=True))
        eids = lax.broadcasted_iota(jnp.int32, (n_tok, N_EXP), 1)
        m0 = (eids == idx_ref[:, 0:1]).astype(jnp.float32)
        m1 = (eids == idx_ref[:, 1:2]).astype(jnp.float32)
        g0 = jnp.sum(p * m0, axis=1, keepdims=True)
        g1 = jnp.sum(p * m1, axis=1, keepdims=True)
        inv = 1.0 / (g0 + g1)
        gates = m0 * (g0 * inv) + m1 * (g1 * inv)

        xb = xf.astype(jnp.bfloat16)

        def contrib(w, dev, k, acc):
            e = dev * E_PER + k
            col = jnp.sum(
                gates * (eids == e).astype(jnp.float32),
                axis=1, keepdims=True)
            yv = lax.dot_general(
                xb, w, (((1,), (0,)), ((), ())),
                preferred_element_type=jnp.float32)
            return acc + col * yv

        acc = jnp.zeros((n_tok, d_ff), jnp.float32)
        for c in range(E_PER):
            acc = contrib(pair_ref[c], my, c, acc)

        for c in range(E_PER):
            xpair_rdma[c].wait_recv()
        for c in range(E_PER):
            cw_rdma[(0, 2 + c)] = lane_hop(cw_ref, cw_s, cw_r, 0, 2 + c,
                                           nxt, xpair_ref.at[c])
            cw_rdma[(0, 2 + c)].start()
            ccw_rdma[(0, 2 + c)] = lane_hop(ccw_ref, ccw_s, ccw_r, 0, 2 + c,
                                            prv, xpair_ref.at[c])
            ccw_rdma[(0, 2 + c)].start()
            started += [cw_rdma[(0, 2 + c)], ccw_rdma[(0, 2 + c)]]
        for c in range(E_PER):
            acc = contrib(xpair_ref[c], pm, c, acc)

        xcw_rdma = {}
        xccw_rdma = {}

        for h in range(CW_H):
            for c in range(E_PER):
                cw_rdma[(h, c)].wait_recv()
                if h + 1 < CW_H:
                    d = lane_hop(cw_ref, cw_s, cw_r, h + 1, c, nxt,
                                 cw_ref.at[h, c])
                    d.start()
                    cw_rdma[(h + 1, c)] = d
                    started.append(d)
                if h >= DUP:
                    d = xrelay(cw_ref, xcw_ref, xcw_s, xcw_r, h, c)
                    d.start()
                    xcw_rdma[(h, c)] = d
                    started.append(d)
            if h < DUP:
                for c in range(E_PER):
                    cw_rdma[(h, 2 + c)].wait_recv()
                    if h + 1 < DUP:
                        d = lane_hop(cw_ref, cw_s, cw_r, h + 1, 2 + c, nxt,
                                     cw_ref.at[h, 2 + c])
                        d.start()
                        cw_rdma[(h + 1, 2 + c)] = d
                        started.append(d)
            if h < CCW_H:
                for c in range(E_PER):
                    ccw_rdma[(h, c)].wait_recv()
                    if h + 1 < CCW_H:
                        d = lane_hop(ccw_ref, ccw_s, ccw_r, h + 1, c, prv,
                                     ccw_ref.at[h, c])
                        d.start()
                        ccw_rdma[(h + 1, c)] = d
                        started.append(d)
                    if h >= DUP:
                        d = xrelay(ccw_ref, xccw_ref, xccw_s, xccw_r, h, c)
                        d.start()
                        xccw_rdma[(h, c)] = d
                        started.append(d)
                if h < DUP:
                    for c in range(E_PER):
                        ccw_rdma[(h, 2 + c)].wait_recv()
                        if h + 1 < min(DUP, CCW_H):
                            d = lane_hop(ccw_ref, ccw_s, ccw_r,
                                         h + 1, 2 + c, prv,
                                         ccw_ref.at[h, 2 + c])
                            d.start()
                            ccw_rdma[(h + 1, 2 + c)] = d
                            started.append(d)

            lo_cw = lane_dev(P - 1 - h)
            lp_cw = pair_dev(P - 1 - h)
            for c in range(E_PER):
                acc = contrib(cw_ref[h, c], lo_cw, c, acc)
            if h < DUP:
                for c in range(E_PER):
                    acc = contrib(cw_ref[h, 2 + c], lp_cw, c, acc)
            else:
                for c in range(E_PER):
                    xcw_rdma[(h, c)].wait_recv()
                    acc = contrib(xcw_ref[h, c], lp_cw, c, acc)
            if h < CCW_H:
                lo_ccw = lane_dev(P + 1 + h)
                lp_ccw = pair_dev(P + 1 + h)
                for c in range(E_PER):
                    acc = contrib(ccw_ref[h, c], lo_ccw, c, acc)
                if h < DUP:
                    for c in range(E_PER):
                        acc = contrib(ccw_ref[h, 2 + c], lp_ccw, c, acc)
                else:
                    for c in range(E_PER):
                        xccw_rdma[(h, c)].wait_recv()
                        acc = contrib(xccw_ref[h, c], lp_ccw, c, acc)

        out_ref[:, :] = acc

        for d in started:
            d.wait_send()

    return pl.pallas_call(
        body,
        out_shape=jax.ShapeDtypeStruct((n_tok, d_ff), jnp.float32),
        in_specs=[pl.BlockSpec(memory_space=pltpu.VMEM)] * 4,
        out_specs=pl.BlockSpec(memory_space=pltpu.VMEM),
        scratch_shapes=[
            pltpu.VMEM((E_PER, d_model, d_ff), jnp.bfloat16),
            pltpu.VMEM((E_PER, d_model, d_ff), jnp.bfloat16),
            pltpu.VMEM((CW_H, 4, d_model, d_ff), jnp.bfloat16),
            pltpu.VMEM((CCW_H, 4, d_model, d_ff), jnp.bfloat16),
            pltpu.VMEM((CW_H, E_PER, d_model, d_ff), jnp.bfloat16),
            pltpu.VMEM((CCW_H, E_PER, d_model, d_ff), jnp.bfloat16),
            pltpu.SemaphoreType.DMA((E_PER,)),
            pltpu.SemaphoreType.DMA((E_PER,)),
            pltpu.SemaphoreType.DMA((CW_H, 4)),
            pltpu.SemaphoreType.DMA((CW_H, 4)),
            pltpu.SemaphoreType.DMA((CCW_H, 4)),
            pltpu.SemaphoreType.DMA((CCW_H, 4)),
            pltpu.SemaphoreType.DMA((CW_H, E_PER)),
            pltpu.SemaphoreType.DMA((CW_H, E_PER)),
            pltpu.SemaphoreType.DMA((CCW_H, E_PER)),
            pltpu.SemaphoreType.DMA((CCW_H, E_PER)),
        ],
        compiler_params=pltpu.CompilerParams(collective_id=0),
    )(x, router_W, route_idx, expert_W)
